# Initial kernel scaffold; baseline (speedup 1.0000x reference)
import jax
import jax.numpy as jnp
from jax.experimental import pallas as pl


def kernel(action_trajectory, style_mu, style_logvar, predicted_transition_count, ground_truth):
    raise NotImplementedError("write your pallas kernel here")


# speedup vs baseline: 14.9702x; 14.9702x over previous
import jax
import jax.numpy as jnp
from jax.experimental import pallas as pl
from jax.experimental.pallas import tpu as pltpu

_GAMMA = 0.1
_BAND = 150
_BIG = 1e10
_KL_FREE_BITS = 0.5
_W_KL, _W_RECON, _W_AUX, _W_TRANS = 1.0, 1.0, 0.1, 0.5


def _softmin3(a, b, c, gamma):
    m = jnp.minimum(jnp.minimum(a, b), c)
    s = (jnp.exp((m - a) * (1.0 / gamma))
         + jnp.exp((m - b) * (1.0 / gamma))
         + jnp.exp((m - c) * (1.0 / gamma)))
    return m - gamma * jnp.log(s)


def _dp_kernel(x_ref, y_ref, o_ref, d1_ref, d2_ref, ys_ref):
    N = x_ref.shape[2]
    R = 48
    d1_ref[...] = jnp.full((R, N), _BIG, jnp.float32)
    d2_ref[...] = jnp.full((R, N), _BIG, jnp.float32)
    ys_ref[...] = jnp.zeros((192, N), jnp.float32)

    x = x_ref[0]
    ii = jax.lax.broadcasted_iota(jnp.int32, (R, N), 1)
    ii4 = jax.lax.broadcasted_iota(jnp.int32, (192, N), 1)

    def step(p, _):
        newcol = jnp.sum(jnp.where(ii4 == p, y_ref[0], 0.0), axis=1,
                         keepdims=True)
        ys = jnp.concatenate([newcol, ys_ref[:, :N - 1]], axis=1)
        ys_ref[...] = ys
        ad = jnp.abs(x - ys)
        Dp = ad[0:R] + ad[R:2 * R] + ad[2 * R:3 * R] + ad[3 * R:4 * R]
        d1 = d1_ref[...]
        d2 = d2_ref[...]
        big_col = jnp.full((R, 1), _BIG, jnp.float32)
        sh_d1 = jnp.concatenate([big_col, d1[:, :N - 1]], axis=1)
        sh_d2 = jnp.concatenate([big_col, d2[:, :N - 1]], axis=1)
        diag_n = jnp.where((ii == 0) & (p == 0), 0.0, sh_d2)
        r = Dp + _softmin3(diag_n, sh_d1, d1, _GAMMA)
        jj = p - ii
        valid = (jj >= 0) & (jj < N) & (jnp.abs(ii - jj) <= _BAND)
        d_cur = jnp.where(valid, r, _BIG)
        d2_ref[...] = d1
        d1_ref[...] = d_cur
        return 0

    jax.lax.fori_loop(0, 2 * N - 1, step, 0)
    o_ref[0] = d1_ref[:, N - 128:]


def _loss_kernel(v_ref, mu_ref, lv_ref, ptc_ref, gtt_ref, att_ref, o_ref):
    v = v_ref[...]
    vnorm = v[0:32] - 0.5 * (v[32:64] + v[64:96])
    recon = jnp.sum(vnorm) * (1.0 / (32.0 * 32.0))

    mu = mu_ref[...]
    lv = lv_ref[...]
    kl_div = -0.5 * jnp.sum(1.0 + lv - mu * mu - jnp.exp(lv), axis=1)
    kl = jnp.mean(jnp.maximum(kl_div - _KL_FREE_BITS, 0.0))

    gtt = gtt_ref[...]
    gt_trans = jnp.sum(jnp.abs(gtt[:, 1:] - gtt[:, :-1]), axis=1)
    ptc = ptc_ref[...][:, 0]
    aux = jnp.mean((ptc - gt_trans) ** 2)

    att = jax.nn.sigmoid((att_ref[...] - 0.5) * 10.0)
    pred_soft = jnp.sum(jnp.abs(att[:, 1:] - att[:, :-1]), axis=1)
    trans = jnp.mean((pred_soft - gt_trans) ** 2)

    total = _W_RECON * recon + _W_KL * kl + _W_AUX * aux + _W_TRANS * trans
    lane = jax.lax.broadcasted_iota(jnp.int32, (1, 8), 1)
    out = (jnp.where(lane == 0, total, 0.0)
           + jnp.where(lane == 1, recon, 0.0)
           + jnp.where(lane == 2, kl, 0.0)
           + jnp.where(lane == 3, aux, 0.0)
           + jnp.where(lane == 4, trans, 0.0))
    o_ref[...] = out


def kernel(action_trajectory, style_mu, style_logvar,
           predicted_transition_count, ground_truth, interpret=False):
    at = action_trajectory
    gt = ground_truth
    B, N, D = at.shape
    xs = jnp.concatenate([at, at, gt], axis=0)
    ys = jnp.concatenate([gt, at, gt], axis=0)
    xc = xs.transpose(2, 0, 1).reshape(D, 2, 48, N).transpose(1, 0, 2, 3)
    xc = xc.reshape(2, 4 * 48, N)
    yc = ys.transpose(2, 0, 1).reshape(D, 2, 48, N).transpose(1, 0, 2, 3)
    yc = yc.reshape(2, 4 * 48, N)

    last = pl.pallas_call(
        _dp_kernel,
        grid=(2,),
        in_specs=[
            pl.BlockSpec((1, 192, N), lambda h: (h, 0, 0)),
            pl.BlockSpec((1, 192, N), lambda h: (h, 0, 0)),
        ],
        out_specs=pl.BlockSpec((1, 48, 128), lambda h: (h, 0, 0)),
        out_shape=jax.ShapeDtypeStruct((2, 48, 128), jnp.float32),
        scratch_shapes=[
            pltpu.VMEM((48, N), jnp.float32),
            pltpu.VMEM((48, N), jnp.float32),
            pltpu.VMEM((192, N), jnp.float32),
        ],
        compiler_params=pltpu.CompilerParams(
            dimension_semantics=("parallel",),
        ),
        name="sdtw_dp",
        interpret=interpret,
    )(xc, yc)
    v = last[:, :, -1].reshape(96, 1)

    out = pl.pallas_call(
        _loss_kernel,
        out_shape=jax.ShapeDtypeStruct((1, 8), jnp.float32),
        name="cvae_losses",
        interpret=interpret,
    )(v, style_mu, style_logvar, predicted_transition_count,
      ground_truth[..., 2], action_trajectory[..., 2])
    return out[0, :5]
